# baseline (speedup 1.0000x reference)
import functools

import jax
import jax.numpy as jnp
from jax import lax
from jax.experimental import pallas as pl
from jax.experimental.pallas import tpu as pltpu
from jax.experimental.pallas import tpu_sc as plsc

N = 10000
E = 320000
D = 128

NC = 2
NS = 16
BLK = 64
NBUF = 4
EPT = 20480
E_PAD = EPT * NS
SB = 32
SBE = SB * BLK
NSB = EPT // SBE
QPSB = SB // NBUF
NPAD = 10112
ROWS_PER_SUB = NPAD // NS


def _sc_agg(x, g1, s1, g2, s2, zeros):
    mesh = plsc.VectorSubcoreMesh(core_axis_name="c", subcore_axis_name="s")

    @functools.partial(
        pl.kernel,
        out_type=[
            jax.ShapeDtypeStruct((NPAD, D), jnp.float32),
            jax.ShapeDtypeStruct((NPAD, D), jnp.float32),
        ],
        mesh=mesh,
        scratch_types=[
            pltpu.VMEM((SBE,), jnp.int32),
            pltpu.VMEM((SBE,), jnp.int32),
            pltpu.VMEM((SBE,), jnp.int32),
            pltpu.VMEM((SBE,), jnp.int32),
            [pltpu.VMEM((BLK, D), jnp.float32) for _ in range(NBUF)],
            pltpu.VMEM_SHARED((NPAD, D), jnp.float32),
            [pltpu.SemaphoreType.DMA for _ in range(NBUF)],
            [pltpu.SemaphoreType.DMA for _ in range(NBUF)],
        ],
    )
    def agg_kernel(x_hbm, g1_hbm, s1_hbm, g2_hbm, s2_hbm, zeros_hbm,
                   agg1_hbm, agg2_hbm,
                   gsa, ssa, gsb, ssb, rows, acc_sp, sem_g, sem_s):
        c = lax.axis_index("c")
        s = lax.axis_index("s")

        def gather(gs, e, b):
            return pltpu.async_copy(
                x_hbm.at[gs.at[pl.ds(e, BLK)]], rows[b], sem_g[b])

        def gather_wait(gs, e, b):
            pltpu.make_async_copy(
                x_hbm.at[gs.at[pl.ds(e, BLK)]], rows[b], sem_g[b]).wait()

        def scatter(ss, e, b):
            return pltpu.async_copy(
                rows[b], acc_sp.at[ss.at[pl.ds(e, BLK)]], sem_s[b],
                add=True)

        def scatter_wait(ss, e, b):
            pltpu.make_async_copy(
                rows[b], acc_sp.at[ss.at[pl.ds(e, BLK)]], sem_s[b]).wait()

        def run_direction(gather_hbm, scatter_hbm, out_hbm):
            base_e = s * EPT

            pltpu.sync_copy(gather_hbm.at[pl.ds(base_e, SBE)], gsa)
            pltpu.sync_copy(scatter_hbm.at[pl.ds(base_e, SBE)], ssa)
            for b in range(NBUF):
                gather(gsa, b * BLK, b)
            pltpu.sync_copy(
                zeros_hbm.at[pl.ds(s * ROWS_PER_SUB, ROWS_PER_SUB)],
                acc_sp.at[pl.ds(s * ROWS_PER_SUB, ROWS_PER_SUB)])
            plsc.subcore_barrier()

            def process_sb(gs_cur, ss_cur, gs_nxt, last_ref):
                def quad(q, carry):
                    for b in range(NBUF):
                        e = (q * NBUF + b) * BLK
                        gather_wait(gs_cur, e, b)
                        scatter(ss_cur, e, b)
                    for b in range(NBUF):
                        e = (q * NBUF + b) * BLK
                        scatter_wait(ss_cur, e, b)

                        @pl.when(q + 1 < QPSB)
                        def _():
                            gather(gs_cur, ((q + 1) * NBUF + b) * BLK, b)

                        @pl.when(jnp.logical_and(q + 1 == QPSB,
                                                 jnp.logical_not(last_ref)))
                        def _():
                            gather(gs_nxt, b * BLK, b)
                    return carry

                lax.fori_loop(0, QPSB, quad, 0)

            def m_body(m, carry):
                off_b = base_e + (2 * m + 1) * SBE
                pltpu.sync_copy(gather_hbm.at[pl.ds(off_b, SBE)], gsb)
                pltpu.sync_copy(scatter_hbm.at[pl.ds(off_b, SBE)], ssb)
                process_sb(gsa, ssa, gsb, jnp.bool_(False))

                @pl.when(m + 1 < NSB // 2)
                def _():
                    off_a = base_e + (2 * m + 2) * SBE
                    pltpu.sync_copy(gather_hbm.at[pl.ds(off_a, SBE)], gsa)
                    pltpu.sync_copy(scatter_hbm.at[pl.ds(off_a, SBE)], ssa)

                process_sb(gsb, ssb, gsa, m + 1 >= NSB // 2)
                return carry

            lax.fori_loop(0, NSB // 2, m_body, 0)
            plsc.subcore_barrier()
            pltpu.sync_copy(acc_sp.at[pl.ds(s * ROWS_PER_SUB, ROWS_PER_SUB)],
                            out_hbm.at[pl.ds(s * ROWS_PER_SUB, ROWS_PER_SUB)])

        @pl.when(c == 0)
        def _():
            run_direction(g1_hbm, s1_hbm, agg1_hbm)

        @pl.when(c == 1)
        def _():
            run_direction(g2_hbm, s2_hbm, agg2_hbm)

    return agg_kernel(x, g1, s1, g2, s2, zeros)


def _tc_mlp_bn(x, agg1, agg2, W1a, b1a, W2a, b2a, W1b, b1b, W2b, b2b,
               gamma, beta):
    def body(x_ref, a1_ref, a2_ref, w1a, b1a_, w2a, b2a_, w1b, b1b_, w2b,
             b2b_, g_ref, be_ref, o_ref):
        h1 = x_ref[...] + a1_ref[...]
        h2 = x_ref[...] + a2_ref[...]
        t1 = jnp.maximum(
            jnp.dot(h1, w1a[...], preferred_element_type=jnp.float32)
            + b1a_[...], 0.0)
        o1 = jnp.maximum(
            jnp.dot(t1, w2a[...], preferred_element_type=jnp.float32)
            + b2a_[...], 0.0)
        t2 = jnp.maximum(
            jnp.dot(h2, w1b[...], preferred_element_type=jnp.float32)
            + b1b_[...], 0.0)
        o2 = jnp.maximum(
            jnp.dot(t2, w2b[...], preferred_element_type=jnp.float32)
            + b2b_[...], 0.0)
        out = (o1 + o2) * 0.5
        mean = jnp.mean(out, axis=0, keepdims=True)
        var = jnp.mean((out - mean) ** 2, axis=0, keepdims=True)
        o_ref[...] = ((out - mean) * lax.rsqrt(var + 1e-5) * g_ref[...]
                      + be_ref[...])

    full = lambda shape: pl.BlockSpec(shape, lambda i: (0,) * len(shape))
    return pl.pallas_call(
        body,
        grid=(1,),
        out_shape=jax.ShapeDtypeStruct((N, D), jnp.float32),
        in_specs=[full((N, D)), full((N, D)), full((N, D)),
                  full((D, D)), full((1, D)), full((D, D)), full((1, D)),
                  full((D, D)), full((1, D)), full((D, D)), full((1, D)),
                  full((1, D)), full((1, D))],
        out_specs=full((N, D)),
    )(x, agg1, agg2, W1a, b1a.reshape(1, D), W2a, b2a.reshape(1, D),
      W1b, b1b.reshape(1, D), W2b, b2b.reshape(1, D),
      gamma.reshape(1, D), beta.reshape(1, D))


def kernel(x, edge_index, W1a, b1a, W2a, b2a, W1b, b1b, W2b, b2b, gamma,
           beta):
    src = edge_index[0].astype(jnp.int32)
    dst = edge_index[1].astype(jnp.int32)
    npad = E_PAD - E
    pad_g = (jnp.arange(npad, dtype=jnp.int32) * 131) % N
    pad_s = N + (jnp.arange(npad, dtype=jnp.int32) % (NPAD - N))
    g1 = jnp.concatenate([src, pad_g])
    s1 = jnp.concatenate([dst, pad_s])
    g2 = jnp.concatenate([dst, pad_g])
    s2 = jnp.concatenate([src, pad_s])
    zeros = jnp.zeros((NPAD, D), jnp.float32)
    agg1, agg2 = _sc_agg(x, g1, s1, g2, s2, zeros)
    return _tc_mlp_bn(x, agg1, agg2, W1a, b1a, W2a, b2a, W1b, b1b, W2b,
                      b2b, gamma, beta)


# speedup vs baseline: 1.0921x; 1.0921x over previous
import functools

import jax
import jax.numpy as jnp
from jax import lax
from jax.experimental import pallas as pl
from jax.experimental.pallas import tpu as pltpu
from jax.experimental.pallas import tpu_sc as plsc

N = 10000
E = 320000
D = 128

NC = 2
NS = 16
BLK = 96
NBUF = 3
NB = 216
EPT = NB * BLK
E_PAD = EPT * NS
SB = 18
SBE = SB * BLK
NSB = NB // SB
TPSB = SB // NBUF
NPAD = 10112
ROWS_PER_SUB = NPAD // NS


def _sc_agg(x, g1, s1, g2, s2, zeros):
    mesh = plsc.VectorSubcoreMesh(core_axis_name="c", subcore_axis_name="s")

    @functools.partial(
        pl.kernel,
        out_type=[
            jax.ShapeDtypeStruct((NPAD, D), jnp.float32),
            jax.ShapeDtypeStruct((NPAD, D), jnp.float32),
        ],
        mesh=mesh,
        scratch_types=[
            pltpu.VMEM((SBE,), jnp.int32),
            pltpu.VMEM((SBE,), jnp.int32),
            pltpu.VMEM((SBE,), jnp.int32),
            pltpu.VMEM((SBE,), jnp.int32),
            [pltpu.VMEM((BLK, D), jnp.float32) for _ in range(NBUF)],
            pltpu.VMEM_SHARED((NPAD, D), jnp.float32),
            [pltpu.SemaphoreType.DMA for _ in range(NBUF)],
            [pltpu.SemaphoreType.DMA for _ in range(NBUF)],
        ],
    )
    def agg_kernel(x_hbm, g1_hbm, s1_hbm, g2_hbm, s2_hbm, zeros_hbm,
                   agg1_hbm, agg2_hbm,
                   gsa, ssa, gsb, ssb, rows, acc_sp, sem_g, sem_s):
        c = lax.axis_index("c")
        s = lax.axis_index("s")

        def gather(gs, e, b):
            pltpu.async_copy(
                x_hbm.at[gs.at[pl.ds(e, BLK)]], rows[b], sem_g[b])

        def wait_g(gs, e, b):
            pltpu.make_async_copy(
                x_hbm.at[gs.at[pl.ds(e, BLK)]], rows[b], sem_g[b]).wait()

        def scatter(ss, e, b):
            pltpu.async_copy(
                rows[b], acc_sp.at[ss.at[pl.ds(e, BLK)]], sem_s[b],
                add=True)

        def wait_s(ss, e, b):
            pltpu.make_async_copy(
                rows[b], acc_sp.at[ss.at[pl.ds(e, BLK)]], sem_s[b]).wait()

        def run_direction(gather_hbm, scatter_hbm, out_hbm):
            base_e = s * EPT

            pltpu.sync_copy(gather_hbm.at[pl.ds(base_e, SBE)], gsa)
            pltpu.sync_copy(scatter_hbm.at[pl.ds(base_e, SBE)], ssa)
            gather(gsa, 0, 0)
            gather(gsa, BLK, 1)
            pltpu.sync_copy(
                zeros_hbm.at[pl.ds(s * ROWS_PER_SUB, ROWS_PER_SUB)],
                acc_sp.at[pl.ds(s * ROWS_PER_SUB, ROWS_PER_SUB)])
            plsc.subcore_barrier()

            def process_sb(cur_g, cur_s, nxt_g, nxt_s, prv_s, wait_first,
                           last):
                def triple(t, carry):
                    for p in range(NBUF):
                        b = p
                        e = (3 * t + p) * BLK
                        wait_g(cur_g, e, b)
                        scatter(cur_s, e, b)
                        bm = (p + 2) % 3
                        if p == 0:
                            @pl.when(jnp.logical_and(t == 0, wait_first))
                            def _():
                                wait_s(prv_s, (SB - 1) * BLK, bm)

                            @pl.when(t > 0)
                            def _():
                                wait_s(cur_s, (3 * t - 1) * BLK, bm)
                        else:
                            wait_s(cur_s, (3 * t + p - 1) * BLK, bm)
                        if p == 0:
                            gather(cur_g, (3 * t + 2) * BLK, bm)
                        else:
                            @pl.when(t < TPSB - 1)
                            def _():
                                gather(cur_g, (3 * t + p + 2) * BLK, bm)

                            @pl.when(jnp.logical_and(
                                t == TPSB - 1, jnp.logical_not(last)))
                            def _():
                                gather(nxt_g, (p + 2 - NBUF) * BLK, bm)
                    return carry

                lax.fori_loop(0, TPSB, triple, 0)

            def m_body(m, carry):
                @pl.when(m > 0)
                def _():
                    wait_s(ssb, (SB - 1) * BLK, 2)

                off_b = base_e + (2 * m + 1) * SBE
                pltpu.sync_copy(gather_hbm.at[pl.ds(off_b, SBE)], gsb)
                pltpu.sync_copy(scatter_hbm.at[pl.ds(off_b, SBE)], ssb)
                process_sb(gsa, ssa, gsb, ssb, ssb, jnp.bool_(False),
                           jnp.bool_(False))

                @pl.when(m + 1 < NSB // 2)
                def _():
                    wait_s(ssa, (SB - 1) * BLK, 2)
                    off_a = base_e + (2 * m + 2) * SBE
                    pltpu.sync_copy(gather_hbm.at[pl.ds(off_a, SBE)], gsa)
                    pltpu.sync_copy(scatter_hbm.at[pl.ds(off_a, SBE)], ssa)

                process_sb(gsb, ssb, gsa, ssa, ssa, m + 1 >= NSB // 2,
                           m + 1 >= NSB // 2)
                return carry

            lax.fori_loop(0, NSB // 2, m_body, 0)

            wait_s(ssb, (SB - 1) * BLK, 2)

            plsc.subcore_barrier()
            pltpu.sync_copy(acc_sp.at[pl.ds(s * ROWS_PER_SUB, ROWS_PER_SUB)],
                            out_hbm.at[pl.ds(s * ROWS_PER_SUB, ROWS_PER_SUB)])

        @pl.when(c == 0)
        def _():
            run_direction(g1_hbm, s1_hbm, agg1_hbm)

        @pl.when(c == 1)
        def _():
            run_direction(g2_hbm, s2_hbm, agg2_hbm)

    return agg_kernel(x, g1, s1, g2, s2, zeros)


def _tc_mlp_bn(x, agg1, agg2, W1a, b1a, W2a, b2a, W1b, b1b, W2b, b2b,
               gamma, beta):
    def body(x_ref, a1_ref, a2_ref, w1a, b1a_, w2a, b2a_, w1b, b1b_, w2b,
             b2b_, g_ref, be_ref, o_ref):
        h1 = x_ref[...] + a1_ref[...]
        h2 = x_ref[...] + a2_ref[...]
        t1 = jnp.maximum(
            jnp.dot(h1, w1a[...], preferred_element_type=jnp.float32)
            + b1a_[...], 0.0)
        o1 = jnp.maximum(
            jnp.dot(t1, w2a[...], preferred_element_type=jnp.float32)
            + b2a_[...], 0.0)
        t2 = jnp.maximum(
            jnp.dot(h2, w1b[...], preferred_element_type=jnp.float32)
            + b1b_[...], 0.0)
        o2 = jnp.maximum(
            jnp.dot(t2, w2b[...], preferred_element_type=jnp.float32)
            + b2b_[...], 0.0)
        out = (o1 + o2) * 0.5
        mean = jnp.mean(out, axis=0, keepdims=True)
        var = jnp.mean((out - mean) ** 2, axis=0, keepdims=True)
        o_ref[...] = ((out - mean) * lax.rsqrt(var + 1e-5) * g_ref[...]
                      + be_ref[...])

    full = lambda shape: pl.BlockSpec(shape, lambda i: (0,) * len(shape))
    return pl.pallas_call(
        body,
        grid=(1,),
        out_shape=jax.ShapeDtypeStruct((N, D), jnp.float32),
        in_specs=[full((N, D)), full((N, D)), full((N, D)),
                  full((D, D)), full((1, D)), full((D, D)), full((1, D)),
                  full((D, D)), full((1, D)), full((D, D)), full((1, D)),
                  full((1, D)), full((1, D))],
        out_specs=full((N, D)),
    )(x, agg1, agg2, W1a, b1a.reshape(1, D), W2a, b2a.reshape(1, D),
      W1b, b1b.reshape(1, D), W2b, b2b.reshape(1, D),
      gamma.reshape(1, D), beta.reshape(1, D))


def kernel(x, edge_index, W1a, b1a, W2a, b2a, W1b, b1b, W2b, b2b, gamma,
           beta):
    src = edge_index[0].astype(jnp.int32)
    dst = edge_index[1].astype(jnp.int32)
    npad = E_PAD - E
    pad_g = (jnp.arange(npad, dtype=jnp.int32) * 131) % N
    pad_s = N + (jnp.arange(npad, dtype=jnp.int32) % (NPAD - N))
    g1 = jnp.concatenate([src, pad_g])
    s1 = jnp.concatenate([dst, pad_s])
    g2 = jnp.concatenate([dst, pad_g])
    s2 = jnp.concatenate([src, pad_s])
    zeros = jnp.zeros((NPAD, D), jnp.float32)
    agg1, agg2 = _sc_agg(x, g1, s1, g2, s2, zeros)
    return _tc_mlp_bn(x, agg1, agg2, W1a, b1a, W2a, b2a, W1b, b1b, W2b,
                      b2b, gamma, beta)


# speedup vs baseline: 1.1027x; 1.0097x over previous
import functools

import jax
import jax.numpy as jnp
from jax import lax
from jax.experimental import pallas as pl
from jax.experimental.pallas import tpu as pltpu
from jax.experimental.pallas import tpu_sc as plsc

N = 10000
E = 320000
D = 128

NC = 2
NS = 16
BLK = 96
NBUF = 3
NB = 210
EPT = NB * BLK
E_PAD = EPT * NS
SB = 15
SBE = SB * BLK
NSB = NB // SB
TPSB = SB // NBUF
NPAD = 10112
ROWS_PER_SUB = NPAD // NS


def _sc_agg(x, g1, s1, g2, s2, zeros):
    mesh = plsc.VectorSubcoreMesh(core_axis_name="c", subcore_axis_name="s")

    @functools.partial(
        pl.kernel,
        out_type=[
            jax.ShapeDtypeStruct((NPAD, D), jnp.float32),
            jax.ShapeDtypeStruct((NPAD, D), jnp.float32),
        ],
        mesh=mesh,
        scratch_types=[
            pltpu.VMEM((SBE,), jnp.int32),
            pltpu.VMEM((SBE,), jnp.int32),
            pltpu.VMEM((SBE,), jnp.int32),
            pltpu.VMEM((SBE,), jnp.int32),
            [pltpu.VMEM((BLK, D), jnp.float32) for _ in range(NBUF)],
            pltpu.VMEM_SHARED((NPAD, D), jnp.float32),
            [pltpu.SemaphoreType.DMA for _ in range(NBUF)],
            [pltpu.SemaphoreType.DMA for _ in range(NBUF)],
        ],
    )
    def agg_kernel(x_hbm, g1_hbm, s1_hbm, g2_hbm, s2_hbm, zeros_hbm,
                   agg1_hbm, agg2_hbm,
                   gsa, ssa, gsb, ssb, rows, acc_sp, sem_g, sem_s):
        c = lax.axis_index("c")
        s = lax.axis_index("s")

        def gather(gs, e, b):
            pltpu.async_copy(
                x_hbm.at[gs.at[pl.ds(e, BLK)]], rows[b], sem_g[b])

        def wait_g(gs, e, b):
            pltpu.make_async_copy(
                x_hbm.at[gs.at[pl.ds(e, BLK)]], rows[b], sem_g[b]).wait()

        def scatter(ss, e, b):
            pltpu.async_copy(
                rows[b], acc_sp.at[ss.at[pl.ds(e, BLK)]], sem_s[b],
                add=True)

        def wait_s(ss, e, b):
            pltpu.make_async_copy(
                rows[b], acc_sp.at[ss.at[pl.ds(e, BLK)]], sem_s[b]).wait()

        def run_direction(gather_hbm, scatter_hbm, out_hbm):
            base_e = s * EPT

            pltpu.sync_copy(gather_hbm.at[pl.ds(base_e, SBE)], gsa)
            pltpu.sync_copy(scatter_hbm.at[pl.ds(base_e, SBE)], ssa)
            gather(gsa, 0, 0)
            gather(gsa, BLK, 1)
            pltpu.sync_copy(
                zeros_hbm.at[pl.ds(s * ROWS_PER_SUB, ROWS_PER_SUB)],
                acc_sp.at[pl.ds(s * ROWS_PER_SUB, ROWS_PER_SUB)])
            plsc.subcore_barrier()

            def process_sb(cur_g, cur_s, nxt_g, nxt_s, prv_s, wait_first,
                           last):
                def triple(t, carry):
                    for p in range(NBUF):
                        b = p
                        e = (3 * t + p) * BLK
                        wait_g(cur_g, e, b)
                        scatter(cur_s, e, b)
                        bm = (p + 2) % 3
                        if p == 0:
                            @pl.when(jnp.logical_and(t == 0, wait_first))
                            def _():
                                wait_s(prv_s, (SB - 1) * BLK, bm)

                            @pl.when(t > 0)
                            def _():
                                wait_s(cur_s, (3 * t - 1) * BLK, bm)
                        else:
                            wait_s(cur_s, (3 * t + p - 1) * BLK, bm)
                        if p == 0:
                            gather(cur_g, (3 * t + 2) * BLK, bm)
                        else:
                            @pl.when(t < TPSB - 1)
                            def _():
                                gather(cur_g, (3 * t + p + 2) * BLK, bm)

                            @pl.when(jnp.logical_and(
                                t == TPSB - 1, jnp.logical_not(last)))
                            def _():
                                gather(nxt_g, (p + 2 - NBUF) * BLK, bm)
                    return carry

                lax.fori_loop(0, TPSB, triple, 0)

            def m_body(m, carry):
                @pl.when(m > 0)
                def _():
                    wait_s(ssb, (SB - 1) * BLK, 2)

                off_b = base_e + (2 * m + 1) * SBE
                pltpu.sync_copy(gather_hbm.at[pl.ds(off_b, SBE)], gsb)
                pltpu.sync_copy(scatter_hbm.at[pl.ds(off_b, SBE)], ssb)
                process_sb(gsa, ssa, gsb, ssb, ssb, jnp.bool_(False),
                           jnp.bool_(False))

                @pl.when(m + 1 < NSB // 2)
                def _():
                    wait_s(ssa, (SB - 1) * BLK, 2)
                    off_a = base_e + (2 * m + 2) * SBE
                    pltpu.sync_copy(gather_hbm.at[pl.ds(off_a, SBE)], gsa)
                    pltpu.sync_copy(scatter_hbm.at[pl.ds(off_a, SBE)], ssa)

                process_sb(gsb, ssb, gsa, ssa, ssa, m + 1 >= NSB // 2,
                           m + 1 >= NSB // 2)
                return carry

            lax.fori_loop(0, NSB // 2, m_body, 0)

            wait_s(ssb, (SB - 1) * BLK, 2)

            plsc.subcore_barrier()
            pltpu.sync_copy(acc_sp.at[pl.ds(s * ROWS_PER_SUB, ROWS_PER_SUB)],
                            out_hbm.at[pl.ds(s * ROWS_PER_SUB, ROWS_PER_SUB)])

        @pl.when(c == 0)
        def _():
            run_direction(g1_hbm, s1_hbm, agg1_hbm)

        @pl.when(c == 1)
        def _():
            run_direction(g2_hbm, s2_hbm, agg2_hbm)

    return agg_kernel(x, g1, s1, g2, s2, zeros)


def _tc_mlp_bn(x, agg1, agg2, W1a, b1a, W2a, b2a, W1b, b1b, W2b, b2b,
               gamma, beta):
    def body(x_ref, a1_ref, a2_ref, w1a, b1a_, w2a, b2a_, w1b, b1b_, w2b,
             b2b_, g_ref, be_ref, o_ref):
        h1 = x_ref[...] + a1_ref[...]
        h2 = x_ref[...] + a2_ref[...]
        t1 = jnp.maximum(
            jnp.dot(h1, w1a[...], preferred_element_type=jnp.float32)
            + b1a_[...], 0.0)
        o1 = jnp.maximum(
            jnp.dot(t1, w2a[...], preferred_element_type=jnp.float32)
            + b2a_[...], 0.0)
        t2 = jnp.maximum(
            jnp.dot(h2, w1b[...], preferred_element_type=jnp.float32)
            + b1b_[...], 0.0)
        o2 = jnp.maximum(
            jnp.dot(t2, w2b[...], preferred_element_type=jnp.float32)
            + b2b_[...], 0.0)
        out = (o1 + o2) * 0.5
        mean = jnp.mean(out, axis=0, keepdims=True)
        var = jnp.mean((out - mean) ** 2, axis=0, keepdims=True)
        o_ref[...] = ((out - mean) * lax.rsqrt(var + 1e-5) * g_ref[...]
                      + be_ref[...])

    full = lambda shape: pl.BlockSpec(shape, lambda i: (0,) * len(shape))
    return pl.pallas_call(
        body,
        grid=(1,),
        out_shape=jax.ShapeDtypeStruct((N, D), jnp.float32),
        in_specs=[full((N, D)), full((N, D)), full((N, D)),
                  full((D, D)), full((1, D)), full((D, D)), full((1, D)),
                  full((D, D)), full((1, D)), full((D, D)), full((1, D)),
                  full((1, D)), full((1, D))],
        out_specs=full((N, D)),
    )(x, agg1, agg2, W1a, b1a.reshape(1, D), W2a, b2a.reshape(1, D),
      W1b, b1b.reshape(1, D), W2b, b2b.reshape(1, D),
      gamma.reshape(1, D), beta.reshape(1, D))


def kernel(x, edge_index, W1a, b1a, W2a, b2a, W1b, b1b, W2b, b2b, gamma,
           beta):
    src = edge_index[0].astype(jnp.int32)
    dst = edge_index[1].astype(jnp.int32)
    npad = E_PAD - E
    pad_g = (jnp.arange(npad, dtype=jnp.int32) * 131) % N
    pad_s = N + (jnp.arange(npad, dtype=jnp.int32) % (NPAD - N))
    g1 = jnp.concatenate([src, pad_g])
    s1 = jnp.concatenate([dst, pad_s])
    g2 = jnp.concatenate([dst, pad_g])
    s2 = jnp.concatenate([src, pad_s])
    zeros = jnp.zeros((NPAD, D), jnp.float32)
    agg1, agg2 = _sc_agg(x, g1, s1, g2, s2, zeros)
    return _tc_mlp_bn(x, agg1, agg2, W1a, b1a, W2a, b2a, W1b, b1b, W2b,
                      b2b, gamma, beta)


# speedup vs baseline: 1.1736x; 1.0643x over previous
import functools

import jax
import jax.numpy as jnp
from jax import lax
from jax.experimental import pallas as pl
from jax.experimental.pallas import tpu as pltpu
from jax.experimental.pallas import tpu_sc as plsc

N = 10000
E = 320000
D = 128

NC = 2
NS = 16
BLK = 96
NBUF = 3
NB = 210
EPT = NB * BLK
E_PAD = EPT * NS
SB = 21
SBE = SB * BLK
NSB = NB // SB
TPSB = SB // NBUF
NPAD = 10112
ROWS_PER_SUB = NPAD // NS


def _sc_agg(x, g1, s1, g2, s2, zeros):
    mesh = plsc.VectorSubcoreMesh(core_axis_name="c", subcore_axis_name="s")

    @functools.partial(
        pl.kernel,
        out_type=[
            jax.ShapeDtypeStruct((NPAD, D), jnp.float32),
            jax.ShapeDtypeStruct((NPAD, D), jnp.float32),
        ],
        mesh=mesh,
        scratch_types=[
            pltpu.VMEM((SBE,), jnp.int32),
            pltpu.VMEM((SBE,), jnp.int32),
            pltpu.VMEM((SBE,), jnp.int32),
            pltpu.VMEM((SBE,), jnp.int32),
            [pltpu.VMEM((BLK, D), jnp.float32) for _ in range(NBUF)],
            pltpu.VMEM_SHARED((NPAD, D), jnp.float32),
            [pltpu.SemaphoreType.DMA for _ in range(NBUF)],
            [pltpu.SemaphoreType.DMA for _ in range(NBUF)],
            pltpu.SemaphoreType.DMA,
        ],
    )
    def agg_kernel(x_hbm, g1_hbm, s1_hbm, g2_hbm, s2_hbm, zeros_hbm,
                   agg1_hbm, agg2_hbm,
                   gsa, ssa, gsb, ssb, rows, acc_sp, sem_g, sem_s,
                   stg):
        c = lax.axis_index("c")
        s = lax.axis_index("s")

        def gather(gs, e, b):
            pltpu.async_copy(
                x_hbm.at[gs.at[pl.ds(e, BLK)]], rows[b], sem_g[b])

        def wait_g(gs, e, b):
            pltpu.make_async_copy(
                x_hbm.at[gs.at[pl.ds(e, BLK)]], rows[b], sem_g[b]).wait()

        def scatter(ss, e, b):
            pltpu.async_copy(
                rows[b], acc_sp.at[ss.at[pl.ds(e, BLK)]], sem_s[b],
                add=True)

        def wait_s(ss, e, b):
            pltpu.make_async_copy(
                rows[b], acc_sp.at[ss.at[pl.ds(e, BLK)]], sem_s[b]).wait()

        def wait_staging(gather_hbm, scatter_hbm, g_buf, s_buf):
            pltpu.make_async_copy(
                gather_hbm.at[pl.ds(0, SBE)], g_buf, stg).wait()
            pltpu.make_async_copy(
                scatter_hbm.at[pl.ds(0, SBE)], s_buf, stg).wait()

        def run_direction(gather_hbm, scatter_hbm, out_hbm):
            base_e = s * EPT

            pltpu.sync_copy(gather_hbm.at[pl.ds(base_e, SBE)], gsa)
            pltpu.sync_copy(scatter_hbm.at[pl.ds(base_e, SBE)], ssa)
            gather(gsa, 0, 0)
            gather(gsa, BLK, 1)
            pltpu.sync_copy(
                zeros_hbm.at[pl.ds(s * ROWS_PER_SUB, ROWS_PER_SUB)],
                acc_sp.at[pl.ds(s * ROWS_PER_SUB, ROWS_PER_SUB)])
            plsc.subcore_barrier()

            def process_sb(gather_hbm, scatter_hbm, cur_g, cur_s, nxt_g,
                           nxt_s, prv_s, wait_first, last):
                def triple(t, carry):
                    @pl.when(jnp.logical_and(t == TPSB - 1,
                                             jnp.logical_not(last)))
                    def _():
                        wait_staging(gather_hbm, scatter_hbm, nxt_g, nxt_s)

                    for p in range(NBUF):
                        b = p
                        e = (3 * t + p) * BLK
                        wait_g(cur_g, e, b)
                        scatter(cur_s, e, b)
                        bm = (p + 2) % 3
                        if p == 0:
                            @pl.when(jnp.logical_and(t == 0, wait_first))
                            def _():
                                wait_s(prv_s, (SB - 1) * BLK, bm)

                            @pl.when(t > 0)
                            def _():
                                wait_s(cur_s, (3 * t - 1) * BLK, bm)
                        else:
                            wait_s(cur_s, (3 * t + p - 1) * BLK, bm)
                        if p == 0:
                            gather(cur_g, (3 * t + 2) * BLK, bm)
                        else:
                            @pl.when(t < TPSB - 1)
                            def _():
                                gather(cur_g, (3 * t + p + 2) * BLK, bm)

                            @pl.when(jnp.logical_and(
                                t == TPSB - 1, jnp.logical_not(last)))
                            def _():
                                gather(nxt_g, (p + 2 - NBUF) * BLK, bm)
                    return carry

                lax.fori_loop(0, TPSB, triple, 0)

            def m_body(m, carry):
                @pl.when(m > 0)
                def _():
                    wait_s(ssb, (SB - 1) * BLK, 2)

                off_b = base_e + (2 * m + 1) * SBE
                pltpu.async_copy(gather_hbm.at[pl.ds(off_b, SBE)], gsb, stg)
                pltpu.async_copy(scatter_hbm.at[pl.ds(off_b, SBE)], ssb, stg)
                process_sb(gather_hbm, scatter_hbm, gsa, ssa, gsb, ssb,
                           ssb, jnp.bool_(False), jnp.bool_(False))

                @pl.when(m + 1 < NSB // 2)
                def _():
                    wait_s(ssa, (SB - 1) * BLK, 2)
                    off_a = base_e + (2 * m + 2) * SBE
                    pltpu.async_copy(
                        gather_hbm.at[pl.ds(off_a, SBE)], gsa, stg)
                    pltpu.async_copy(
                        scatter_hbm.at[pl.ds(off_a, SBE)], ssa, stg)

                process_sb(gather_hbm, scatter_hbm, gsb, ssb, gsa, ssa,
                           ssa, m + 1 >= NSB // 2, m + 1 >= NSB // 2)
                return carry

            lax.fori_loop(0, NSB // 2, m_body, 0)

            wait_s(ssb, (SB - 1) * BLK, 2)

            plsc.subcore_barrier()
            pltpu.sync_copy(acc_sp.at[pl.ds(s * ROWS_PER_SUB, ROWS_PER_SUB)],
                            out_hbm.at[pl.ds(s * ROWS_PER_SUB, ROWS_PER_SUB)])

        @pl.when(c == 0)
        def _():
            run_direction(g1_hbm, s1_hbm, agg1_hbm)

        @pl.when(c == 1)
        def _():
            run_direction(g2_hbm, s2_hbm, agg2_hbm)

    return agg_kernel(x, g1, s1, g2, s2, zeros)


def _tc_mlp_bn(x, agg1, agg2, W1a, b1a, W2a, b2a, W1b, b1b, W2b, b2b,
               gamma, beta):
    def body(x_ref, a1_ref, a2_ref, w1a, b1a_, w2a, b2a_, w1b, b1b_, w2b,
             b2b_, g_ref, be_ref, o_ref):
        h1 = x_ref[...] + a1_ref[...]
        h2 = x_ref[...] + a2_ref[...]
        t1 = jnp.maximum(
            jnp.dot(h1, w1a[...], preferred_element_type=jnp.float32)
            + b1a_[...], 0.0)
        o1 = jnp.maximum(
            jnp.dot(t1, w2a[...], preferred_element_type=jnp.float32)
            + b2a_[...], 0.0)
        t2 = jnp.maximum(
            jnp.dot(h2, w1b[...], preferred_element_type=jnp.float32)
            + b1b_[...], 0.0)
        o2 = jnp.maximum(
            jnp.dot(t2, w2b[...], preferred_element_type=jnp.float32)
            + b2b_[...], 0.0)
        out = (o1 + o2) * 0.5
        mean = jnp.mean(out, axis=0, keepdims=True)
        var = jnp.mean((out - mean) ** 2, axis=0, keepdims=True)
        o_ref[...] = ((out - mean) * lax.rsqrt(var + 1e-5) * g_ref[...]
                      + be_ref[...])

    full = lambda shape: pl.BlockSpec(shape, lambda i: (0,) * len(shape))
    return pl.pallas_call(
        body,
        grid=(1,),
        out_shape=jax.ShapeDtypeStruct((N, D), jnp.float32),
        in_specs=[full((N, D)), full((N, D)), full((N, D)),
                  full((D, D)), full((1, D)), full((D, D)), full((1, D)),
                  full((D, D)), full((1, D)), full((D, D)), full((1, D)),
                  full((1, D)), full((1, D))],
        out_specs=full((N, D)),
    )(x, agg1, agg2, W1a, b1a.reshape(1, D), W2a, b2a.reshape(1, D),
      W1b, b1b.reshape(1, D), W2b, b2b.reshape(1, D),
      gamma.reshape(1, D), beta.reshape(1, D))


def kernel(x, edge_index, W1a, b1a, W2a, b2a, W1b, b1b, W2b, b2b, gamma,
           beta):
    src = edge_index[0].astype(jnp.int32)
    dst = edge_index[1].astype(jnp.int32)
    npad = E_PAD - E
    pad_g = (jnp.arange(npad, dtype=jnp.int32) * 131) % N
    pad_s = N + (jnp.arange(npad, dtype=jnp.int32) % (NPAD - N))
    g1 = jnp.concatenate([src, pad_g])
    s1 = jnp.concatenate([dst, pad_s])
    g2 = jnp.concatenate([dst, pad_g])
    s2 = jnp.concatenate([src, pad_s])
    zeros = jnp.zeros((NPAD, D), jnp.float32)
    agg1, agg2 = _sc_agg(x, g1, s1, g2, s2, zeros)
    return _tc_mlp_bn(x, agg1, agg2, W1a, b1a, W2a, b2a, W1b, b1b, W2b,
                      b2b, gamma, beta)
